# baseline (speedup 1.0000x reference)
import jax
import jax.numpy as jnp
from jax import lax
from jax.experimental import pallas as pl
from jax.experimental.pallas import tpu as pltpu
from jax.experimental.pallas import tpu_sc as plsc

DIM = 33
NC, NS, L = 2, 16, 16
NW = NC * NS
B, C, H, W = 8, 3, 512, 512
PLANE = H * W
NPIX = B * PLANE
PER_W = NPIX // NW
CHUNK = 1024
NCHUNK = PER_W // CHUNK
NPAIR = NCHUNK // 2

RP = DIM - 1
SG = RP
SB = DIM * RP
SCH = DIM * DIM * RP
LUT_WORDS = 3 * SCH
SPAN = 33760


ROWS = CHUNK // W


def _body(lut_hbm, x_hbm, out_hbm, lut_v, inb, outb, sin0, sin1, sout0, sout1):
    wid = lax.axis_index("c") * NS + lax.axis_index("s")
    batch = wid // 4
    quarter = wid % 4
    row_base = quarter * (PER_W // W)

    pltpu.sync_copy(lut_hbm, lut_v)

    scale = jnp.float32(DIM - 1)
    sins = (sin0, sin1)
    souts = (sout0, sout1)

    def vslot(ref, par, c):
        return ref.at[par * 3 + c]

    def start_in(par, ci):
        r0 = row_base + ci * ROWS
        for c in range(3):
            pltpu.async_copy(x_hbm.at[batch, c, pl.ds(r0, ROWS), :],
                             vslot(inb, par, c), sins[par])

    def wait_in(par, ci):
        r0 = row_base + ci * ROWS
        for c in range(3):
            pltpu.make_async_copy(x_hbm.at[batch, c, pl.ds(r0, ROWS), :],
                                  vslot(inb, par, c), sins[par]).wait()

    def start_out(par, ci):
        r0 = row_base + ci * ROWS
        for c in range(3):
            pltpu.async_copy(vslot(outb, par, c),
                             out_hbm.at[batch, c, pl.ds(r0, ROWS), :], souts[par])

    def wait_out(par, ci):
        r0 = row_base + ci * ROWS
        for c in range(3):
            pltpu.make_async_copy(vslot(outb, par, c),
                                  out_hbm.at[batch, c, pl.ds(r0, ROWS), :],
                                  souts[par]).wait()

    def compute(par):
        rbuf = vslot(inb, par, 0)
        gbuf = vslot(inb, par, 1)
        bbuf = vslot(inb, par, 2)
        orb = vslot(outb, par, 0)
        ogb = vslot(outb, par, 1)
        obb = vslot(outb, par, 2)

        @plsc.parallel_loop(0, CHUNK // L, unroll=6)
        def inner(i):
            row = i // (W // L)
            s = pl.multiple_of((i % (W // L)) * L, L)
            r = rbuf[row, pl.ds(s, L)]
            g = gbuf[row, pl.ds(s, L)]
            b = bbuf[row, pl.ds(s, L)]
            pr = r * scale
            pg = g * scale
            pb = b * scale
            ir = pr.astype(jnp.int32)
            ig = pg.astype(jnp.int32)
            ib = pb.astype(jnp.int32)
            fr = pr - ir.astype(jnp.float32)
            fg = pg - ig.astype(jnp.float32)
            fb = pb - ib.astype(jnp.float32)
            idx0 = ib * SB + ig * SG + ir

            def corner(ofs):
                return plsc.load_gather(lut_v.at[pl.ds(ofs, SPAN)], [idx0])

            def channel(cofs):
                w00 = corner(cofs)
                w01 = corner(cofs + SG)
                w10 = corner(cofs + SB)
                w11 = corner(cofs + SB + SG)

                def rlerp(wf):
                    wv = plsc.bitcast(wf, jnp.int32)
                    lo = plsc.bitcast(wv << 16, jnp.float32)
                    d = plsc.bitcast(wv & jnp.int32(-65536), jnp.float32)
                    return lo + fr * d

                v00 = rlerp(w00)
                v01 = rlerp(w01)
                v10 = rlerp(w10)
                v11 = rlerp(w11)
                u0 = v00 + fg * (v01 - v00)
                u1 = v10 + fg * (v11 - v10)
                return u0 + fb * (u1 - u0)

            orb[row, pl.ds(s, L)] = channel(0)
            ogb[row, pl.ds(s, L)] = channel(SCH)
            obb[row, pl.ds(s, L)] = channel(2 * SCH)

    start_in(0, 0)

    def pair_body(cp, _):
        c0 = 2 * cp
        c1 = c0 + 1
        start_in(1, c1)

        @pl.when(cp > 0)
        def _():
            wait_out(0, c0 - 2)

        wait_in(0, c0)
        compute(0)
        start_out(0, c0)

        @pl.when(cp < NPAIR - 1)
        def _():
            start_in(0, c0 + 2)

        @pl.when(cp > 0)
        def _():
            wait_out(1, c1 - 2)

        wait_in(1, c1)
        compute(1)
        start_out(1, c1)
        return 0

    lax.fori_loop(0, NPAIR, pair_body, 0, unroll=False)
    wait_out(0, NCHUNK - 2)
    wait_out(1, NCHUNK - 1)


def _lut_apply(packed_lut, x_flat):
    mesh = plsc.VectorSubcoreMesh(
        core_axis_name="c", subcore_axis_name="s", num_cores=NC, num_subcores=NS
    )
    f = pl.kernel(
        _body,
        out_type=jax.ShapeDtypeStruct((B, C, H, W), jnp.float32),
        mesh=mesh,
        scratch_types=[
            pltpu.VMEM((LUT_WORDS,), jnp.float32),
            pltpu.VMEM((6, ROWS, W), jnp.float32),
            pltpu.VMEM((6, ROWS, W), jnp.float32),
            pltpu.SemaphoreType.DMA,
            pltpu.SemaphoreType.DMA,
            pltpu.SemaphoreType.DMA,
            pltpu.SemaphoreType.DMA,
        ],
        compiler_params=pltpu.CompilerParams(
            needs_layout_passes=False, use_tc_tiling_on_sc=True
        ),
    )
    return f(packed_lut, x_flat)


def kernel(LUT, x):
    lo = LUT[..., : DIM - 1]
    delta = LUT[..., 1:] - lo
    lo16 = lax.bitcast_convert_type(lo.astype(jnp.bfloat16), jnp.uint16)
    d16 = lax.bitcast_convert_type(delta.astype(jnp.bfloat16), jnp.uint16)
    words = lo16.astype(jnp.uint32) | (d16.astype(jnp.uint32) << 16)
    packed = lax.bitcast_convert_type(words, jnp.float32).reshape(-1)
    return _lut_apply(packed, x)


# speedup vs baseline: 1.2407x; 1.2407x over previous
import jax
import jax.numpy as jnp
from jax import lax
from jax.experimental import pallas as pl
from jax.experimental.pallas import tpu as pltpu
from jax.experimental.pallas import tpu_sc as plsc

DIM = 33
NC, NS, L = 2, 16, 16
NW = NC * NS
B, C, H, W = 8, 3, 512, 512
PLANE = H * W
NPIX = B * PLANE
PER_W = NPIX // NW
CHUNK = 1024
NCHUNK = PER_W // CHUNK
NPAIR = NCHUNK // 2

RP = DIM - 1
SG = RP
SB = DIM * RP
SCH = DIM * DIM * RP
LUT_WORDS = 3 * SCH
SPAN = 33760


ROWS = CHUNK // W


def _body(lut_hbm, x_hbm, out_hbm, lut_v, inb, outb, sin0, sin1, sout0, sout1):
    wid = lax.axis_index("c") * NS + lax.axis_index("s")
    batch = wid // 4
    quarter = wid % 4
    row_base = quarter * (PER_W // W)

    pltpu.sync_copy(lut_hbm, lut_v)

    scale = jnp.float32(DIM - 1)
    sins = (sin0, sin1)
    souts = (sout0, sout1)

    def vslot(ref, par, c):
        return ref.at[par * 3 + c]

    def start_in(par, ci):
        r0 = row_base + ci * ROWS
        for c in range(3):
            pltpu.async_copy(x_hbm.at[batch, c, pl.ds(r0, ROWS), :],
                             vslot(inb, par, c), sins[par])

    def wait_in(par, ci):
        r0 = row_base + ci * ROWS
        for c in range(3):
            pltpu.make_async_copy(x_hbm.at[batch, c, pl.ds(r0, ROWS), :],
                                  vslot(inb, par, c), sins[par]).wait()

    def start_out(par, ci):
        r0 = row_base + ci * ROWS
        for c in range(3):
            pltpu.async_copy(vslot(outb, par, c),
                             out_hbm.at[batch, c, pl.ds(r0, ROWS), :], souts[par])

    def wait_out(par, ci):
        r0 = row_base + ci * ROWS
        for c in range(3):
            pltpu.make_async_copy(vslot(outb, par, c),
                                  out_hbm.at[batch, c, pl.ds(r0, ROWS), :],
                                  souts[par]).wait()

    def compute(par):
        rbuf = vslot(inb, par, 0)
        gbuf = vslot(inb, par, 1)
        bbuf = vslot(inb, par, 2)
        orb = vslot(outb, par, 0)
        ogb = vslot(outb, par, 1)
        obb = vslot(outb, par, 2)

        @plsc.parallel_loop(0, CHUNK // L, unroll=3)
        def inner(i):
            row = i // (W // L)
            s = pl.multiple_of((i % (W // L)) * L, L)
            r = rbuf[row, pl.ds(s, L)]
            g = gbuf[row, pl.ds(s, L)]
            b = bbuf[row, pl.ds(s, L)]
            pr = r * scale
            pg = g * scale
            pb = b * scale
            ir = pr.astype(jnp.int32)
            ig = pg.astype(jnp.int32)
            ib = pb.astype(jnp.int32)
            fr = pr - ir.astype(jnp.float32)
            fg = pg - ig.astype(jnp.float32)
            fb = pb - ib.astype(jnp.float32)
            idx0 = ib * SB + ig * SG + ir

            def corner(ofs):
                return plsc.load_gather(lut_v.at[pl.ds(ofs, SPAN)], [idx0])

            def channel(cofs):
                w00 = corner(cofs)
                w01 = corner(cofs + SG)
                w10 = corner(cofs + SB)
                w11 = corner(cofs + SB + SG)

                def rlerp(wf):
                    wv = plsc.bitcast(wf, jnp.int32)
                    lo = plsc.bitcast(wv << 16, jnp.float32)
                    d = plsc.bitcast(wv & jnp.int32(-65536), jnp.float32)
                    return lo + fr * d

                v00 = rlerp(w00)
                v01 = rlerp(w01)
                v10 = rlerp(w10)
                v11 = rlerp(w11)
                u0 = v00 + fg * (v01 - v00)
                u1 = v10 + fg * (v11 - v10)
                return u0 + fb * (u1 - u0)

            orb[row, pl.ds(s, L)] = channel(0)
            ogb[row, pl.ds(s, L)] = channel(SCH)
            obb[row, pl.ds(s, L)] = channel(2 * SCH)

    start_in(0, 0)

    def pair_body(cp, _):
        c0 = 2 * cp
        c1 = c0 + 1
        start_in(1, c1)

        @pl.when(cp > 0)
        def _():
            wait_out(0, c0 - 2)

        wait_in(0, c0)
        compute(0)
        start_out(0, c0)

        @pl.when(cp < NPAIR - 1)
        def _():
            start_in(0, c0 + 2)

        @pl.when(cp > 0)
        def _():
            wait_out(1, c1 - 2)

        wait_in(1, c1)
        compute(1)
        start_out(1, c1)
        return 0

    lax.fori_loop(0, NPAIR, pair_body, 0, unroll=False)
    wait_out(0, NCHUNK - 2)
    wait_out(1, NCHUNK - 1)


def _lut_apply(packed_lut, x_flat):
    mesh = plsc.VectorSubcoreMesh(
        core_axis_name="c", subcore_axis_name="s", num_cores=NC, num_subcores=NS
    )
    f = pl.kernel(
        _body,
        out_type=jax.ShapeDtypeStruct((B, C, H, W), jnp.float32),
        mesh=mesh,
        scratch_types=[
            pltpu.VMEM((LUT_WORDS,), jnp.float32),
            pltpu.VMEM((6, ROWS, W), jnp.float32),
            pltpu.VMEM((6, ROWS, W), jnp.float32),
            pltpu.SemaphoreType.DMA,
            pltpu.SemaphoreType.DMA,
            pltpu.SemaphoreType.DMA,
            pltpu.SemaphoreType.DMA,
        ],
        compiler_params=pltpu.CompilerParams(
            needs_layout_passes=False, use_tc_tiling_on_sc=True
        ),
    )
    return f(packed_lut, x_flat)


def kernel(LUT, x):
    lo = LUT[..., : DIM - 1]
    delta = LUT[..., 1:] - lo
    lo16 = lax.bitcast_convert_type(lo.astype(jnp.bfloat16), jnp.uint16)
    d16 = lax.bitcast_convert_type(delta.astype(jnp.bfloat16), jnp.uint16)
    words = lo16.astype(jnp.uint32) | (d16.astype(jnp.uint32) << 16)
    packed = lax.bitcast_convert_type(words, jnp.float32).reshape(-1)
    return _lut_apply(packed, x)


# speedup vs baseline: 1.2468x; 1.0049x over previous
import jax
import jax.numpy as jnp
from jax import lax
from jax.experimental import pallas as pl
from jax.experimental.pallas import tpu as pltpu
from jax.experimental.pallas import tpu_sc as plsc

DIM = 33
NC, NS, L = 2, 16, 16
NW = NC * NS
B, C, H, W = 8, 3, 512, 512
PLANE = H * W
NPIX = B * PLANE
PER_W = NPIX // NW
CHUNK = 1024
NCHUNK = PER_W // CHUNK
NPAIR = NCHUNK // 2

RP = DIM - 1
SG = RP
SB = DIM * RP
SCH = DIM * DIM * RP
LUT_WORDS = 3 * SCH
SPAN = 33760


ROWS = CHUNK // W


def _body(lut_hbm, x_hbm, out_hbm, lut_v, inb, outb, sin0, sin1, sout0, sout1):
    wid = lax.axis_index("c") * NS + lax.axis_index("s")
    batch = wid // 4
    quarter = wid % 4
    row_base = quarter * (PER_W // W)

    scale = jnp.float32(DIM - 1)
    sins = (sin0, sin1)
    souts = (sout0, sout1)

    def vslot(ref, par, c):
        return ref.at[par * 3 + c]

    def start_in(par, ci):
        r0 = row_base + ci * ROWS
        for c in range(3):
            pltpu.async_copy(x_hbm.at[batch, c, pl.ds(r0, ROWS), :],
                             vslot(inb, par, c), sins[par])

    def wait_in(par, ci):
        r0 = row_base + ci * ROWS
        for c in range(3):
            pltpu.make_async_copy(x_hbm.at[batch, c, pl.ds(r0, ROWS), :],
                                  vslot(inb, par, c), sins[par]).wait()

    def start_out(par, ci):
        r0 = row_base + ci * ROWS
        for c in range(3):
            pltpu.async_copy(vslot(outb, par, c),
                             out_hbm.at[batch, c, pl.ds(r0, ROWS), :], souts[par])

    def wait_out(par, ci):
        r0 = row_base + ci * ROWS
        for c in range(3):
            pltpu.make_async_copy(vslot(outb, par, c),
                                  out_hbm.at[batch, c, pl.ds(r0, ROWS), :],
                                  souts[par]).wait()

    def compute(par):
        rbuf = vslot(inb, par, 0)
        gbuf = vslot(inb, par, 1)
        bbuf = vslot(inb, par, 2)
        orb = vslot(outb, par, 0)
        ogb = vslot(outb, par, 1)
        obb = vslot(outb, par, 2)

        @plsc.parallel_loop(0, CHUNK // L, unroll=3)
        def inner(i):
            row = i // (W // L)
            s = pl.multiple_of((i % (W // L)) * L, L)
            r = rbuf[row, pl.ds(s, L)]
            g = gbuf[row, pl.ds(s, L)]
            b = bbuf[row, pl.ds(s, L)]
            pr = r * scale
            pg = g * scale
            pb = b * scale
            ir = pr.astype(jnp.int32)
            ig = pg.astype(jnp.int32)
            ib = pb.astype(jnp.int32)
            fr = pr - ir.astype(jnp.float32)
            fg = pg - ig.astype(jnp.float32)
            fb = pb - ib.astype(jnp.float32)
            idx0 = ib * SB + ig * SG + ir

            def corner(ofs):
                return plsc.load_gather(lut_v.at[pl.ds(ofs, SPAN)], [idx0])

            def channel(cofs):
                w00 = corner(cofs)
                w01 = corner(cofs + SG)
                w10 = corner(cofs + SB)
                w11 = corner(cofs + SB + SG)

                def rlerp(wf):
                    wv = plsc.bitcast(wf, jnp.int32)
                    lo = plsc.bitcast(wv << 16, jnp.float32)
                    d = plsc.bitcast(wv & jnp.int32(-65536), jnp.float32)
                    return lo + fr * d

                v00 = rlerp(w00)
                v01 = rlerp(w01)
                v10 = rlerp(w10)
                v11 = rlerp(w11)
                u0 = v00 + fg * (v01 - v00)
                u1 = v10 + fg * (v11 - v10)
                return u0 + fb * (u1 - u0)

            orb[row, pl.ds(s, L)] = channel(0)
            ogb[row, pl.ds(s, L)] = channel(SCH)
            obb[row, pl.ds(s, L)] = channel(2 * SCH)

    start_in(0, 0)
    pltpu.sync_copy(lut_hbm, lut_v)

    def pair_body(cp, _):
        c0 = 2 * cp
        c1 = c0 + 1
        start_in(1, c1)

        @pl.when(cp > 0)
        def _():
            wait_out(0, c0 - 2)

        wait_in(0, c0)
        compute(0)
        start_out(0, c0)

        @pl.when(cp < NPAIR - 1)
        def _():
            start_in(0, c0 + 2)

        @pl.when(cp > 0)
        def _():
            wait_out(1, c1 - 2)

        wait_in(1, c1)
        compute(1)
        start_out(1, c1)
        return 0

    lax.fori_loop(0, NPAIR, pair_body, 0, unroll=False)
    wait_out(0, NCHUNK - 2)
    wait_out(1, NCHUNK - 1)


def _lut_apply(packed_lut, x_flat):
    mesh = plsc.VectorSubcoreMesh(
        core_axis_name="c", subcore_axis_name="s", num_cores=NC, num_subcores=NS
    )
    f = pl.kernel(
        _body,
        out_type=jax.ShapeDtypeStruct((B, C, H, W), jnp.float32),
        mesh=mesh,
        scratch_types=[
            pltpu.VMEM((LUT_WORDS,), jnp.float32),
            pltpu.VMEM((6, ROWS, W), jnp.float32),
            pltpu.VMEM((6, ROWS, W), jnp.float32),
            pltpu.SemaphoreType.DMA,
            pltpu.SemaphoreType.DMA,
            pltpu.SemaphoreType.DMA,
            pltpu.SemaphoreType.DMA,
        ],
        compiler_params=pltpu.CompilerParams(
            needs_layout_passes=False, use_tc_tiling_on_sc=True
        ),
    )
    return f(packed_lut, x_flat)


def kernel(LUT, x):
    lo = LUT[..., : DIM - 1]
    delta = LUT[..., 1:] - lo
    lo16 = lax.bitcast_convert_type(lo.astype(jnp.bfloat16), jnp.uint16)
    d16 = lax.bitcast_convert_type(delta.astype(jnp.bfloat16), jnp.uint16)
    words = lo16.astype(jnp.uint32) | (d16.astype(jnp.uint32) << 16)
    packed = lax.bitcast_convert_type(words, jnp.float32).reshape(-1)
    return _lut_apply(packed, x)
